# Initial kernel scaffold; baseline (speedup 1.0000x reference)
import jax
import jax.numpy as jnp
from jax.experimental import pallas as pl


def kernel(binary_input, W):
    raise NotImplementedError("write your pallas kernel here")


# speedup vs baseline: 9.3303x; 9.3303x over previous
import jax
import jax.numpy as jnp
from jax.experimental import pallas as pl

EMBED = 16
VOCAB = 32
ROW_BLK = 512


def _body(x_ref, we_ref, rs_ref, emb_ref, lp_ref):
    amp = (x_ref[...] * 2 - 1).astype(jnp.float32)
    emb_ref[...] = jnp.dot(amp, we_ref[...], preferred_element_type=jnp.float32)
    lp_ref[...] = amp * rs_ref[...]


def kernel(binary_input, W):
    B, S, L = binary_input.shape
    H = W.shape[1]
    R = B * S
    x2d = binary_input.reshape(R, L)
    w_expanded = (jnp.eye(L, dtype=W.dtype)[:, :, None] * W[None, :, :]).reshape(L, L * H)
    rowsum = jnp.sum(W, axis=1, keepdims=True).T

    grid = (R // ROW_BLK,)
    emb2d, lp2d = pl.pallas_call(
        _body,
        grid=grid,
        in_specs=[
            pl.BlockSpec((ROW_BLK, L), lambda i: (i, 0)),
            pl.BlockSpec((L, L * H), lambda i: (0, 0)),
            pl.BlockSpec((1, L), lambda i: (0, 0)),
        ],
        out_specs=[
            pl.BlockSpec((ROW_BLK, L * H), lambda i: (i, 0)),
            pl.BlockSpec((ROW_BLK, L), lambda i: (i, 0)),
        ],
        out_shape=[
            jax.ShapeDtypeStruct((R, L * H), jnp.float32),
            jax.ShapeDtypeStruct((R, L), jnp.float32),
        ],
    )(x2d, w_expanded, rowsum)

    emb = emb2d.reshape(B, S, L, H)
    lp = lp2d.reshape(B, S, L, 1)
    return (emb, lp)


# speedup vs baseline: 111.4635x; 11.9464x over previous
import jax
import jax.numpy as jnp
from jax.experimental import pallas as pl


def _body(x_ref, wexp_ref, w_ref, emb_ref, lp_ref):
    amp = (x_ref[...] * 2 - 1).astype(jnp.float32)
    emb_ref[...] = jnp.dot(wexp_ref[...], amp, preferred_element_type=jnp.float32)
    rs = jnp.sum(w_ref[...], axis=1, keepdims=True)
    lp_ref[...] = amp * rs


def kernel(binary_input, W):
    B, S, L = binary_input.shape
    H = W.shape[1]
    x2 = binary_input.transpose(1, 2, 0).reshape(S * L, B)
    eye = jnp.eye(L, dtype=W.dtype)
    w_expT = (eye[:, None, :] * W[:, :, None]).reshape(L * H, L)

    emb2, lp2 = pl.pallas_call(
        _body,
        grid=(S,),
        in_specs=[
            pl.BlockSpec((L, B), lambda s: (s, 0)),
            pl.BlockSpec((L * H, L), lambda s: (0, 0)),
            pl.BlockSpec((L, H), lambda s: (0, 0)),
        ],
        out_specs=[
            pl.BlockSpec((L * H, B), lambda s: (s, 0)),
            pl.BlockSpec((L, B), lambda s: (s, 0)),
        ],
        out_shape=[
            jax.ShapeDtypeStruct((S * L * H, B), jnp.float32),
            jax.ShapeDtypeStruct((S * L, B), jnp.float32),
        ],
    )(x2, w_expT, W)

    emb = emb2.reshape(S, L, H, B).transpose(3, 0, 1, 2)
    lp = lp2.reshape(S, L, B).transpose(2, 0, 1).reshape(B, S, L, 1)
    return (emb, lp)


# speedup vs baseline: 137.3491x; 1.2322x over previous
import jax
import jax.numpy as jnp
from jax.experimental import pallas as pl

SBLK = 2


def _body(x_ref, wexp_ref, w_ref, emb_ref, lp_ref):
    amp = (x_ref[...] * 2 - 1).astype(jnp.float32)
    emb_ref[...] = jnp.dot(wexp_ref[...], amp, preferred_element_type=jnp.float32)
    rs = jnp.sum(w_ref[...], axis=1, keepdims=True)
    lp_ref[...] = amp * jnp.tile(rs, (SBLK, 1))


def kernel(binary_input, W):
    B, S, L = binary_input.shape
    H = W.shape[1]
    x2 = binary_input.transpose(1, 2, 0).reshape(S * L, B)
    eye = jnp.eye(SBLK * L, dtype=W.dtype)
    w_tile = jnp.tile(W, (SBLK, 1))
    w_expT = (eye[:, None, :] * w_tile[:, :, None]).reshape(SBLK * L * H, SBLK * L)

    emb2, lp2 = pl.pallas_call(
        _body,
        grid=(S // SBLK,),
        in_specs=[
            pl.BlockSpec((SBLK * L, B), lambda s: (s, 0)),
            pl.BlockSpec((SBLK * L * H, SBLK * L), lambda s: (0, 0)),
            pl.BlockSpec((L, H), lambda s: (0, 0)),
        ],
        out_specs=[
            pl.BlockSpec((SBLK * L * H, B), lambda s: (s, 0)),
            pl.BlockSpec((SBLK * L, B), lambda s: (s, 0)),
        ],
        out_shape=[
            jax.ShapeDtypeStruct((S * L * H, B), jnp.float32),
            jax.ShapeDtypeStruct((S * L, B), jnp.float32),
        ],
    )(x2, w_expT, W)

    emb = emb2.reshape(S, L, H, B).transpose(3, 0, 1, 2)
    lp = lp2.reshape(S, L, B).transpose(2, 0, 1).reshape(B, S, L, 1)
    return (emb, lp)


# speedup vs baseline: 150.9794x; 1.0992x over previous
import jax
import jax.numpy as jnp
from jax.experimental import pallas as pl

SBLK = 5


def _body(x_ref, wexp_ref, w_ref, emb_ref, lp_ref):
    L, H = w_ref.shape
    amp = (x_ref[...] * 2 - 1).astype(jnp.float32)
    for sb in range(SBLK):
        emb_ref[sb * L * H:(sb + 1) * L * H, :] = jnp.dot(
            wexp_ref[...], amp[sb * L:(sb + 1) * L, :],
            preferred_element_type=jnp.float32)
    rs = jnp.sum(w_ref[...], axis=1, keepdims=True)
    lp_ref[...] = amp * jnp.tile(rs, (SBLK, 1))


def kernel(binary_input, W):
    B, S, L = binary_input.shape
    H = W.shape[1]
    x2 = binary_input.transpose(1, 2, 0).reshape(S * L, B)
    eye = jnp.eye(L, dtype=W.dtype)
    w_expT = (eye[:, None, :] * W[:, :, None]).reshape(L * H, L)

    emb2, lp2 = pl.pallas_call(
        _body,
        grid=(S // SBLK,),
        in_specs=[
            pl.BlockSpec((SBLK * L, B), lambda s: (s, 0)),
            pl.BlockSpec((L * H, L), lambda s: (0, 0)),
            pl.BlockSpec((L, H), lambda s: (0, 0)),
        ],
        out_specs=[
            pl.BlockSpec((SBLK * L * H, B), lambda s: (s, 0)),
            pl.BlockSpec((SBLK * L, B), lambda s: (s, 0)),
        ],
        out_shape=[
            jax.ShapeDtypeStruct((S * L * H, B), jnp.float32),
            jax.ShapeDtypeStruct((S * L, B), jnp.float32),
        ],
    )(x2, w_expT, W)

    emb = emb2.reshape(S, L, H, B).transpose(3, 0, 1, 2)
    lp = lp2.reshape(S, L, B).transpose(2, 0, 1).reshape(B, S, L, 1)
    return (emb, lp)
